# baseline (speedup 1.0000x reference)
import functools

import jax
import jax.numpy as jnp
from jax import lax
from jax.experimental import pallas as pl
from jax.experimental.pallas import tpu as pltpu
from jax.experimental.pallas import tpu_sc as plsc

_ROWS = 100000
_EMBED = 64
_NW = 32
_CHUNK = 1000
_NCHUNK = _ROWS // _CHUNK
_ROUNDS = -(-_NCHUNK // _NW)


@functools.partial(
    pl.kernel,
    out_type=(
        jax.ShapeDtypeStruct((_ROWS, _EMBED), jnp.float32),
        jax.ShapeDtypeStruct((_ROWS, _EMBED), jnp.float32),
    ),
    mesh=plsc.VectorSubcoreMesh(core_axis_name="c", subcore_axis_name="s"),
    scratch_types=[
        pltpu.VMEM((_CHUNK, _EMBED), jnp.float32),
        pltpu.SemaphoreType.DMA,
        pltpu.SemaphoreType.DMA,
    ],
)
def _sc_copy(u_in, i_in, u_out, i_out, buf, gsem, ssem):
    wid = lax.axis_index("s") * 2 + lax.axis_index("c")

    for r in range(_ROUNDS):
        c = r * _NW + wid
        off = c * _CHUNK

        def _round(off=off):
            sl = pl.ds(off, _CHUNK)
            for src, dst in ((u_in, u_out), (i_in, i_out)):
                g = pltpu.make_async_copy(src.at[sl], buf, gsem)
                g.start()
                g.wait()
                s = pltpu.make_async_copy(buf, dst.at[sl], ssem)
                s.start()
                s.wait()

        if (r + 1) * _NW <= _NCHUNK:
            _round()
        else:
            pl.when(c < _NCHUNK)(_round)


def kernel(user_weight, item_weight):
    return _sc_copy(user_weight, item_weight)


# speedup vs baseline: 1.1319x; 1.1319x over previous
import functools

import jax
import jax.numpy as jnp
from jax import lax
from jax.experimental import pallas as pl
from jax.experimental.pallas import tpu as pltpu
from jax.experimental.pallas import tpu_sc as plsc

_ROWS = 100000
_EMBED = 64


_TC_BLK = 10000


def _tc_copy_kernel(x_in, x_out):
    x_out[...] = x_in[...]


def _tc_copy(x):
    spec = pl.BlockSpec((_TC_BLK, _EMBED), lambda n: (n, 0))
    return pl.pallas_call(
        _tc_copy_kernel,
        grid=(_ROWS // _TC_BLK,),
        out_shape=jax.ShapeDtypeStruct(x.shape, x.dtype),
        in_specs=[spec],
        out_specs=spec,
    )(x)


_NW = 32
_CHUNK = 400
_NCHUNK = _ROWS // _CHUNK
_FULL_ROUNDS = _NCHUNK // _NW
_TAIL = _NCHUNK - _FULL_ROUNDS * _NW


@functools.partial(
    pl.kernel,
    out_type=jax.ShapeDtypeStruct((_ROWS, _EMBED), jnp.float32),
    mesh=plsc.VectorSubcoreMesh(core_axis_name="c", subcore_axis_name="s"),
    scratch_types=[
        pltpu.VMEM((_CHUNK, _EMBED), jnp.float32),
        pltpu.VMEM((_CHUNK, _EMBED), jnp.float32),
        pltpu.SemaphoreType.DMA,
        pltpu.SemaphoreType.DMA,
        pltpu.SemaphoreType.DMA,
        pltpu.SemaphoreType.DMA,
    ],
)
def _sc_copy(x_in, x_out, buf0, buf1, gs0, gs1, ss0, ss1):
    wid = lax.axis_index("s") * 2 + lax.axis_index("c")
    bufs = (buf0, buf1)
    gsem = (gs0, gs1)
    ssem = (ss0, ss1)

    n = _FULL_ROUNDS

    def chunk_slice(j):
        return pl.ds((j * _NW + wid) * _CHUNK, _CHUNK)

    def gather(j):
        return pltpu.make_async_copy(
            x_in.at[chunk_slice(j)], bufs[j % 2], gsem[j % 2]
        )

    def scatter(j):
        return pltpu.make_async_copy(
            bufs[j % 2], x_out.at[chunk_slice(j)], ssem[j % 2]
        )

    gather(0).start()
    gather(1).start()
    for j in range(n):
        gather(j).wait()
        s = scatter(j)
        s.start()
        if j + 2 < n:
            s.wait()
            gather(j + 2).start()
    scatter(n - 2).wait()
    scatter(n - 1).wait()

    def _tail():
        sl = pl.ds((_FULL_ROUNDS * _NW + wid) * _CHUNK, _CHUNK)
        g = pltpu.make_async_copy(x_in.at[sl], bufs[0], gsem[0])
        g.start()
        g.wait()
        s = pltpu.make_async_copy(bufs[0], x_out.at[sl], ssem[0])
        s.start()
        s.wait()

    pl.when(wid < _TAIL)(_tail)


def kernel(user_weight, item_weight):
    return _tc_copy(user_weight), _sc_copy(item_weight)


# speedup vs baseline: 1.1366x; 1.0041x over previous
import functools

import jax
import jax.numpy as jnp
from jax import lax
from jax.experimental import pallas as pl
from jax.experimental.pallas import tpu as pltpu
from jax.experimental.pallas import tpu_sc as plsc

_ROWS = 100000
_EMBED = 64


_TC_BLK = 10000


def _tc_copy_kernel(x_in, x_out):
    x_out[...] = x_in[...]


def _tc_copy(x):
    spec = pl.BlockSpec((_TC_BLK, _EMBED), lambda n: (n, 0))
    return pl.pallas_call(
        _tc_copy_kernel,
        grid=(_ROWS // _TC_BLK,),
        out_shape=jax.ShapeDtypeStruct(x.shape, x.dtype),
        in_specs=[spec],
        out_specs=spec,
    )(x)


_NW = 32
_CHUNK = 200
_NCHUNK = _ROWS // _CHUNK
_FULL_ROUNDS = _NCHUNK // _NW
_TAIL = _NCHUNK - _FULL_ROUNDS * _NW


@functools.partial(
    pl.kernel,
    out_type=jax.ShapeDtypeStruct((_ROWS, _EMBED), jnp.float32),
    mesh=plsc.VectorSubcoreMesh(core_axis_name="c", subcore_axis_name="s"),
    scratch_types=[
        pltpu.VMEM_SHARED((_CHUNK, _EMBED), jnp.float32),
        pltpu.VMEM_SHARED((_CHUNK, _EMBED), jnp.float32),
        pltpu.SemaphoreType.DMA,
        pltpu.SemaphoreType.DMA,
        pltpu.SemaphoreType.DMA,
        pltpu.SemaphoreType.DMA,
    ],
)
def _sc_copy(x_in, x_out, buf0, buf1, gs0, gs1, ss0, ss1):
    wid = lax.axis_index("s") * 2 + lax.axis_index("c")
    bufs = (buf0, buf1)
    gsem = (gs0, gs1)
    ssem = (ss0, ss1)

    n = _FULL_ROUNDS

    def chunk_slice(j):
        return pl.ds((j * _NW + wid) * _CHUNK, _CHUNK)

    def gather(j):
        return pltpu.make_async_copy(
            x_in.at[chunk_slice(j)], bufs[j % 2], gsem[j % 2]
        )

    def scatter(j):
        return pltpu.make_async_copy(
            bufs[j % 2], x_out.at[chunk_slice(j)], ssem[j % 2]
        )

    gather(0).start()
    gather(1).start()
    for j in range(n):
        gather(j).wait()
        s = scatter(j)
        s.start()
        if j + 2 < n:
            s.wait()
            gather(j + 2).start()
    scatter(n - 2).wait()
    scatter(n - 1).wait()

    def _tail():
        sl = pl.ds((_FULL_ROUNDS * _NW + wid) * _CHUNK, _CHUNK)
        g = pltpu.make_async_copy(x_in.at[sl], bufs[0], gsem[0])
        g.start()
        g.wait()
        s = pltpu.make_async_copy(bufs[0], x_out.at[sl], ssem[0])
        s.start()
        s.wait()

    pl.when(wid < _TAIL)(_tail)


def kernel(user_weight, item_weight):
    return _tc_copy(user_weight), _sc_copy(item_weight)
